# baseline (speedup 1.0000x reference)
import functools

import jax
import jax.numpy as jnp
from jax import lax
from jax.experimental import pallas as pl
from jax.experimental.pallas import tpu as pltpu
from jax.experimental.pallas import tpu_sc as plsc

VOCAB = 1000000
B = 16384
L = 50
D = 32
C = 16

NC = 2
NS = 16
NW = NC * NS
RPW = B // NW
CHUNK = 128
NCH = RPW // CHUNK

BN = 8192


def _tc_project(table_t, w):
    def body(t_ref, w_ref, o_ref):
        ws = w_ref[...] * (1.0 / L)
        res = lax.dot_general(
            t_ref[...], ws, (((0,), (1,)), ((), ())),
            preferred_element_type=jnp.float32,
        )
        res3 = res.reshape(BN // 8, 8, C)
        for j in range(8):
            o_ref[:, j * C:(j + 1) * C] = res3[:, j, :]

    return pl.pallas_call(
        body,
        grid=(pl.cdiv(VOCAB, BN),),
        in_specs=[
            pl.BlockSpec((D, BN), lambda i: (0, i)),
            pl.BlockSpec((C, D), lambda i: (0, 0)),
        ],
        out_specs=pl.BlockSpec((BN // 8, 8 * C), lambda i: (i, 0)),
        out_shape=jax.ShapeDtypeStruct((VOCAB * C // 128, 128), jnp.float32),
    )(table_t, w)


def _sc_bag(idx_t, p, bias):
    mesh = plsc.VectorSubcoreMesh(
        core_axis_name="c", subcore_axis_name="s", num_cores=NC, num_subcores=NS
    )

    @functools.partial(
        pl.kernel,
        mesh=mesh,
        out_type=jax.ShapeDtypeStruct((B, C), jnp.float32),
        scratch_types=[
            pltpu.VMEM((L, RPW), jnp.int32),
            pltpu.VMEM((RPW, C), jnp.float32),
            pltpu.VMEM((C,), jnp.float32),
            pltpu.SemaphoreType.DMA,
        ],
        compiler_params=pltpu.CompilerParams(use_tc_tiling_on_sc=False),
    )
    def k(idx_hbm, p_hbm, bias_hbm, out_hbm, idx_v, acc_v, bias_v, sem):
        wid = lax.axis_index("s") * NC + lax.axis_index("c")
        base = wid * RPW
        pltpu.sync_copy(idx_hbm.at[:, pl.ds(base, RPW)], idx_v)
        pltpu.sync_copy(bias_hbm, bias_v)

        zero = jnp.zeros((C,), jnp.float32)

        def zero_row(r, _):
            acc_v[r] = zero
            return 0

        lax.fori_loop(0, RPW, zero_row, 0)

        for c in range(NCH):
            sl = pl.ds(c * CHUNK, CHUNK)
            dst = acc_v.at[pl.ds(c * CHUNK, CHUNK)]

            def fire(j, _):
                pltpu.async_copy(
                    p_hbm.at[idx_v.at[j, sl]], dst, sem, add=True
                )
                return 0

            lax.fori_loop(0, L, fire, 0)

        drain = pltpu.make_async_copy(
            p_hbm.at[pl.ds(0, CHUNK)], acc_v.at[pl.ds(0, CHUNK)], sem
        )

        def drain_one(i, _):
            drain.wait()
            return 0

        lax.fori_loop(0, NCH * L, drain_one, 0)

        bias_vec = bias_v[...]

        def add_bias(r, _):
            acc_v[r] = acc_v[r] + bias_vec
            return 0

        lax.fori_loop(0, RPW, add_bias, 0)
        pltpu.sync_copy(acc_v, out_hbm.at[pl.ds(base, RPW)])

    return k(idx_t, p, bias)


def kernel(text, emb_weight, fc_weight, fc_bias):
    table_t = jnp.swapaxes(emb_weight, 0, 1)
    p_wide = _tc_project(table_t, fc_weight)
    p = jnp.reshape(p_wide, (VOCAB, C))
    idx_t = jnp.swapaxes(text.astype(jnp.int32), 0, 1)
    return _sc_bag(idx_t, p, fc_bias)
